# baseline (speedup 1.0000x reference)
import jax
import jax.numpy as jnp
from jax.experimental import pallas as pl
from jax.experimental.pallas import tpu as pltpu

_SLOT_ROWS = 4096
_NBUF = 6
_WSLACK = 3


def _chunk_schedule(n):
    if n < 2 * _SLOT_ROWS:
        return [n]
    ramp = [512, 512, 1024, 2048]
    head = [r for r in ramp]
    tail = [r for r in reversed(ramp)]
    mid_total = n - sum(head) - sum(tail)
    if mid_total < 0 or mid_total % _SLOT_ROWS != 0:
        return [_SLOT_ROWS] * (n // _SLOT_ROWS) + (
            [n % _SLOT_ROWS] if n % _SLOT_ROWS else [])
    return head + [_SLOT_ROWS] * (mid_total // _SLOT_ROWS) + tail


def _ring_body(x_ref, out_ref, mask_ref, buf, insem, outsem):
    n = x_ref.shape[0]
    sizes = _chunk_schedule(n)
    starts = []
    acc = 0
    for sz in sizes:
        starts.append(acc)
        acc += sz
    nch = len(sizes)

    def in_copy(j):
        b = j % _NBUF
        return pltpu.make_async_copy(
            x_ref.at[pl.ds(starts[j], sizes[j])],
            buf.at[b, pl.ds(0, sizes[j])],
            insem.at[b],
        )

    def out_copy(j):
        b = j % _NBUF
        return pltpu.make_async_copy(
            buf.at[b, pl.ds(0, sizes[j])],
            out_ref.at[pl.ds(starts[j], sizes[j])],
            outsem.at[b],
        )

    for j in range(min(_NBUF, nch)):
        in_copy(j).start()
    mask_ref[...] = jnp.ones(mask_ref.shape, dtype=jnp.bool_)
    for i in range(nch):
        in_copy(i).wait()
        out_copy(i).start()
        if i >= _WSLACK and (i - _WSLACK) + _NBUF < nch:
            out_copy(i - _WSLACK).wait()
            in_copy((i - _WSLACK) + _NBUF).start()
    for i in range(max(0, nch - _NBUF), nch):
        out_copy(i).wait()


def kernel(bags):
    b, s, d = bags.shape
    n = b * s
    flat = bags.reshape(n, d)
    slot = min(_SLOT_ROWS, n)
    padded, mask = pl.pallas_call(
        _ring_body,
        in_specs=[pl.BlockSpec(memory_space=pl.ANY)],
        out_specs=(
            pl.BlockSpec(memory_space=pl.ANY),
            pl.BlockSpec(memory_space=pltpu.MemorySpace.VMEM),
        ),
        out_shape=(
            jax.ShapeDtypeStruct((n, d), bags.dtype),
            jax.ShapeDtypeStruct((b, s), jnp.bool_),
        ),
        scratch_shapes=[
            pltpu.VMEM((_NBUF, slot, d), bags.dtype),
            pltpu.SemaphoreType.DMA((_NBUF,)),
            pltpu.SemaphoreType.DMA((_NBUF,)),
        ],
    )(flat)
    return (padded.reshape(b, s, d), mask)
